# Initial kernel scaffold; baseline (speedup 1.0000x reference)
import jax
import jax.numpy as jnp
from jax.experimental import pallas as pl


def kernel(x, edge_index, matrix, params):
    raise NotImplementedError("write your pallas kernel here")


# speedup vs baseline: 6.9233x; 6.9233x over previous
import functools
import numpy as np
import jax
import jax.numpy as jnp
from jax import lax
from jax.experimental import pallas as pl
from jax.experimental.pallas import tpu as pltpu

N = 10000
E = 320000
D_IN = 128
D_OUT = 128
RATE_NODE = 0.5
NOISE_NODE = 0.05
RATE_EDGE = 0.5
NOISE_EDGE = 0.05
ALPHA = 0.5
SCE_ALPHA = 2.0
N_ADD_MAX = int(NOISE_EDGE * E)
E2 = E + N_ADD_MAX + N


def _const_tables():
    rng = np.random.RandomState(0)
    perm = rng.permutation(N)
    num_mask = int(RATE_NODE * N)
    mask_nodes = perm[:num_mask]
    perm_mask = rng.permutation(num_mask)
    n_noise = int(NOISE_NODE * num_mask)
    token_nodes = mask_nodes[perm_mask[:int((1 - NOISE_NODE) * num_mask)]]
    noise_nodes = mask_nodes[perm_mask[num_mask - n_noise:]]
    noise_chosen = rng.permutation(N)[:n_noise]
    srcmap = np.arange(N, dtype=np.int32)
    srcmap[noise_nodes] = noise_chosen.astype(np.int32)
    token_flag = np.zeros((N,), np.bool_)
    token_flag[token_nodes] = True

    rng1 = np.random.RandomState(1)
    u_keep = rng1.random_sample(E) >= RATE_EDGE
    rng2 = np.random.RandomState(1)
    words = rng2.randint(0, 2 ** 32, size=2 * E + 1000000, dtype=np.uint32)
    mask_bits = (1 << (N - 1).bit_length()) - 1
    vals = (words[1::2] & mask_bits).astype(np.int64)
    acc = vals <= (N - 1)
    accepted = vals[acc].astype(np.int32)
    cumacc = np.concatenate([[0], np.cumsum(acc)]).astype(np.int32)
    accepted = np.concatenate([accepted, np.zeros(2 * N_ADD_MAX, np.int32)])
    return token_flag, srcmap, u_keep, accepted, cumacc


_TOKEN_FLAG, _SRCMAP, _U_KEEP, _ACCEPTED, _CUMACC = _const_tables()


def _mask_edges_fast(src, dst):
    valid = src != dst
    cs = jnp.cumsum(valid.astype(jnp.int32))
    rank = jnp.clip(cs - 1, 0, E - 1)
    keep2 = valid & jnp.asarray(_U_KEEP)[rank]
    w1 = keep2.astype(jnp.float32)
    k = cs[-1]
    m = keep2.sum().astype(jnp.int32)
    n_add = (m * int(NOISE_EDGE * 100)) // 100
    start = jnp.asarray(_CUMACC)[k]
    idx = jnp.arange(N_ADD_MAX, dtype=jnp.int32)
    act = idx < n_add
    acc_t = jnp.asarray(_ACCEPTED)
    add_s = jnp.where(act, acc_t[start + idx], 0)
    add_d = jnp.where(act, acc_t[start + n_add + idx], 0)
    w2 = act.astype(jnp.float32)
    loops = jnp.arange(N, dtype=jnp.int32)
    s = jnp.concatenate([src.astype(jnp.int32), add_s, loops])
    d = jnp.concatenate([dst.astype(jnp.int32), add_d, loops])
    w = jnp.concatenate([w1, w2, jnp.ones((N,), jnp.float32)])
    return s, d, w


def _prelu(x, a):
    return jnp.where(x >= 0, x, a * x)


def _l2norm(x):
    return x / jnp.clip(jnp.linalg.norm(x, axis=-1, keepdims=True), 1e-12, None)


def _bn(h, g, b):
    return (h - h.mean(0)) / jnp.sqrt(h.var(0) + 1e-5) * g + b


def _seg_agg(xs, src, dst, w=None):
    xe = xs[src] if w is None else xs[src] * w[:, None]
    return jax.ops.segment_sum(xe, dst, num_segments=N)


def _degrees(src, dst, w=None):
    if w is None:
        deg_out = jnp.zeros((N,), jnp.float32).at[src].add(1.0)
        deg_in = jnp.zeros((N,), jnp.float32).at[dst].add(1.0)
    else:
        deg_out = jnp.zeros((N,), jnp.float32).at[src].add(w)
        deg_in = jnp.zeros((N,), jnp.float32).at[dst].add(w)
    return jnp.clip(deg_out, 1.0, None), jnp.clip(deg_in, 1.0, None)


def _gconv(x, src, dst, L, norm, w=None, degs=None):
    if norm == 'both':
        deg_out, deg_in = degs
        x = x * (deg_out ** -0.5)[:, None]
    agg = _seg_agg(x, src, dst, w)
    if norm == 'both':
        agg = agg * (deg_in ** -0.5)[:, None]
    return _prelu(agg @ L['W'] + L['b'], L['ac'])


def _encoder(x, src, dst, layers, w=None, degs=None):
    h = x
    for L in layers:
        h = _gconv(h, src, dst, L, 'both', w, degs)
        h = _bn(h, L['g'], L['be'])
        h = _prelu(h, L['aa'])
    return h


def _decoder(x, src, dst, layers, normalize, w=None):
    h = x
    for L in layers:
        h = _gconv(h, src, dst, L, 'none', w)
        h = _bn(h, L['g'], L['be'])
        h = _prelu(h, L['aa'])
    if normalize:
        h = _l2norm(h)
    return h


def _sce(x, y):
    xn = _l2norm(x)
    yn = _l2norm(y)
    return jnp.mean((1.0 - (xn * yn).sum(-1)) ** SCE_ALPHA)


_BI = 400


def _loss2_body(hi_ref, h_ref, a_ref, out_ref):
    p = lax.dot_general(hi_ref[...], h_ref[...], (((1,), (1,)), ((), ())),
                        preferred_element_type=jnp.float32)
    d = p - a_ref[...]
    part = jnp.sum(d * d).reshape(1, 1)

    @pl.when(pl.program_id(0) == 0)
    def _():
        out_ref[...] = jnp.zeros((1, 1), jnp.float32)

    out_ref[...] += part


def _loss2(h2, matrix):
    gi = N // _BI
    total = pl.pallas_call(
        _loss2_body,
        grid=(gi,),
        in_specs=[
            pl.BlockSpec((_BI, D_IN), lambda i: (i, 0)),
            pl.BlockSpec((N, D_IN), lambda i: (0, 0)),
            pl.BlockSpec((_BI, N), lambda i: (i, 0)),
        ],
        out_specs=pl.BlockSpec((1, 1), lambda i: (0, 0)),
        out_shape=jax.ShapeDtypeStruct((1, 1), jnp.float32),
    )(h2, h2, matrix)
    return total[0, 0] / (float(N) * float(N))


def kernel(x, edge_index, matrix, params):
    src, dst = edge_index[0].astype(jnp.int32), edge_index[1].astype(jnp.int32)
    msrc, mdst, mw = _mask_edges_fast(src, dst)

    token = jnp.asarray(_TOKEN_FLAG)[:, None]
    xm = jnp.where(token, params['mask_token'][0][None, :], x[jnp.asarray(_SRCMAP)])

    degs1 = _degrees(src, dst)
    degs2 = _degrees(msrc, mdst, mw)

    h1 = _encoder(xm, src, dst, params['enc'], degs=degs1)
    h2 = _encoder(x, msrc, mdst, params['enc'], mw, degs=degs2)
    h1 = _decoder(h1, src, dst, params['dec1'], False)
    h2 = _decoder(h2, msrc, mdst, params['dec2'], True, mw)

    loss1 = _sce(h1, x)
    loss2 = _loss2(h2, matrix)
    return ALPHA * loss1 + (1.0 - ALPHA) * loss2


# speedup vs baseline: 37.8158x; 5.4621x over previous
import functools
import numpy as np
import jax
import jax.numpy as jnp
from jax import lax
from jax.experimental import pallas as pl
from jax.experimental.pallas import tpu as pltpu
from jax.experimental.pallas import tpu_sc as plsc

N = 10000
E = 320000
D_IN = 128
D_OUT = 128
RATE_NODE = 0.5
NOISE_NODE = 0.05
RATE_EDGE = 0.5
NOISE_EDGE = 0.05
ALPHA = 0.5
SCE_ALPHA = 2.0
N_ADD_MAX = int(NOISE_EDGE * E)
E2 = E + N_ADD_MAX + N

NZPAD = 128
NP = N + NZPAD
NW = 32
BATCH = 128
ROWS_PER_TILE = N // 16


def _edge_batches(num_edges):
    return -(-num_edges // (NW * BATCH))


NB1 = _edge_batches(E)
NB2 = _edge_batches(E2)
EP1 = NW * BATCH * NB1
EP2 = NW * BATCH * NB2


def _const_tables():
    rng = np.random.RandomState(0)
    perm = rng.permutation(N)
    num_mask = int(RATE_NODE * N)
    mask_nodes = perm[:num_mask]
    perm_mask = rng.permutation(num_mask)
    n_noise = int(NOISE_NODE * num_mask)
    token_nodes = mask_nodes[perm_mask[:int((1 - NOISE_NODE) * num_mask)]]
    noise_nodes = mask_nodes[perm_mask[num_mask - n_noise:]]
    noise_chosen = rng.permutation(N)[:n_noise]
    srcmap = np.arange(N, dtype=np.int32)
    srcmap[noise_nodes] = noise_chosen.astype(np.int32)
    token_flag = np.zeros((N,), np.bool_)
    token_flag[token_nodes] = True

    rng1 = np.random.RandomState(1)
    u_keep = rng1.random_sample(E) >= RATE_EDGE
    rng2 = np.random.RandomState(1)
    words = rng2.randint(0, 2 ** 32, size=2 * E + 1000000, dtype=np.uint32)
    mask_bits = (1 << (N - 1).bit_length()) - 1
    vals = (words[1::2] & mask_bits).astype(np.int64)
    acc = vals <= (N - 1)
    accepted = vals[acc].astype(np.int32)
    cumacc = np.concatenate([[0], np.cumsum(acc)]).astype(np.int32)
    accepted = np.concatenate([accepted, np.zeros(2 * N_ADD_MAX, np.int32)])
    return token_flag, srcmap, u_keep, accepted, cumacc


_TOKEN_FLAG, _SRCMAP, _U_KEEP, _ACCEPTED, _CUMACC = _const_tables()


def _mask_edges_fast(src, dst):
    valid = src != dst
    cs = jnp.cumsum(valid.astype(jnp.int32))
    rank = jnp.clip(cs - 1, 0, E - 1)
    keep2 = valid & jnp.asarray(_U_KEEP)[rank]
    w1 = keep2.astype(jnp.float32)
    k = cs[-1]
    m = keep2.sum().astype(jnp.int32)
    n_add = (m * int(NOISE_EDGE * 100)) // 100
    start = jnp.asarray(_CUMACC)[k]
    idx = jnp.arange(N_ADD_MAX, dtype=jnp.int32)
    act = idx < n_add
    acc_t = jnp.asarray(_ACCEPTED)
    add_s = jnp.where(act, acc_t[start + idx], 0)
    add_d = jnp.where(act, acc_t[start + n_add + idx], 0)
    w2 = act.astype(jnp.float32)
    loops = jnp.arange(N, dtype=jnp.int32)
    s = jnp.concatenate([src.astype(jnp.int32), add_s, loops])
    d = jnp.concatenate([dst.astype(jnp.int32), add_d, loops])
    w = jnp.concatenate([w1, w2, jnp.ones((N,), jnp.float32)])
    return s, d, w


def _prelu(x, a):
    return jnp.where(x >= 0, x, a * x)


def _make_agg(nb):
    mesh = plsc.VectorSubcoreMesh(core_axis_name="c", subcore_axis_name="s")

    @functools.partial(
        pl.kernel,
        mesh=mesh,
        out_type=jax.ShapeDtypeStruct((2 * 16, ROWS_PER_TILE, D_IN), jnp.float32),
        scratch_types=[
            pltpu.VMEM((nb, BATCH), jnp.int32),
            pltpu.VMEM((nb, BATCH), jnp.int32),
            pltpu.VMEM((BATCH, D_IN), jnp.float32),
            pltpu.VMEM_SHARED((N, D_IN), jnp.float32),
            pltpu.SemaphoreType.DMA,
        ],
    )
    def agg(table_hbm, srcx_hbm, dstx_hbm, zeros_hbm, out_hbm,
            sidx, didx, rows, acc, sem):
        c = lax.axis_index("c")
        s = lax.axis_index("s")
        wid = s * 2 + c

        pltpu.sync_copy(zeros_hbm, acc.at[pl.ds(s * ROWS_PER_TILE, ROWS_PER_TILE)])
        pltpu.sync_copy(srcx_hbm.at[wid], sidx)
        pltpu.sync_copy(dstx_hbm.at[wid], didx)
        plsc.subcore_barrier()

        for b in range(nb):
            pltpu.async_copy(table_hbm.at[sidx.at[b]], rows, sem).wait()
            pltpu.sync_copy(rows, acc.at[didx.at[b]], add=True)

        plsc.subcore_barrier()
        pltpu.sync_copy(
            acc.at[pl.ds(s * ROWS_PER_TILE, ROWS_PER_TILE)],
            out_hbm.at[c * 16 + s])

    return agg


_AGG1 = _make_agg(NB1)
_AGG2 = _make_agg(NB2)
_AGG_ZEROS = np.zeros((ROWS_PER_TILE, D_IN), np.float32)


def _pad_edges(sx, dx, ep):
    n = sx.shape[0]
    pad = ep - n
    fill = N + (jnp.arange(pad, dtype=jnp.int32) & (NZPAD - 1))
    shape = (NW, ep // (NW * BATCH), BATCH)
    sxp = jnp.concatenate([sx, fill]).reshape(shape)
    dxp = jnp.concatenate([dx, jnp.zeros((pad,), jnp.int32)]).reshape(shape)
    return sxp, dxp


def _agg(table, sxp, dxp, nb):
    f = _AGG1 if nb == NB1 else _AGG2
    out = f(table, sxp, dxp, jnp.asarray(_AGG_ZEROS))
    return out.reshape(2 * N, D_IN)


def _dense_body(tail, part_ref, sin_ref, sout_ref, w_ref, vec_ref, x_ref,
                out_ref):
    agg = part_ref[:N, :] + part_ref[N:, :]
    agg = agg * sin_ref[...]
    ac = vec_ref[3, 0]
    aa = vec_ref[4, 0]
    h = lax.dot_general(agg, w_ref[...], (((1,), (0,)), ((), ())),
                        preferred_element_type=jnp.float32)
    h = _prelu(h + vec_ref[0, :][None, :], ac)
    mu = jnp.mean(h, axis=0, keepdims=True)
    hc = h - mu
    var = jnp.mean(hc * hc, axis=0, keepdims=True)
    h = hc / jnp.sqrt(var + 1e-5) * vec_ref[1, :][None, :] + vec_ref[2, :][None, :]
    h = _prelu(h, aa)
    if tail == 'scale_pad':
        out_ref[:N, :] = h * sout_ref[...]
        out_ref[N:, :] = jnp.zeros((NZPAD, D_IN), jnp.float32)
    elif tail == 'pad':
        out_ref[:N, :] = h
        out_ref[N:, :] = jnp.zeros((NZPAD, D_IN), jnp.float32)
    elif tail == 'l2norm':
        nrm = jnp.sqrt(jnp.sum(h * h, axis=1, keepdims=True))
        out_ref[...] = h / jnp.clip(nrm, 1e-12, None)
    else:
        hn = h / jnp.clip(jnp.sqrt(jnp.sum(h * h, axis=1, keepdims=True)),
                          1e-12, None)
        x = x_ref[...]
        xn = x / jnp.clip(jnp.sqrt(jnp.sum(x * x, axis=1, keepdims=True)),
                          1e-12, None)
        cos = jnp.sum(hn * xn, axis=1)
        out_ref[...] = jnp.mean((1.0 - cos) ** SCE_ALPHA).reshape(1, 1)


def _dense(tail, part, sin, sout, L, x=None):
    if tail == 'sce':
        out_shape = jax.ShapeDtypeStruct((1, 1), jnp.float32)
    elif tail == 'l2norm':
        out_shape = jax.ShapeDtypeStruct((N, D_IN), jnp.float32)
    else:
        out_shape = jax.ShapeDtypeStruct((NP, D_IN), jnp.float32)
    vec = jnp.stack([
        L['b'], L['g'], L['be'],
        jnp.full((D_OUT,), L['ac'], jnp.float32),
        jnp.full((D_OUT,), L['aa'], jnp.float32),
    ])
    if sin is None:
        sin = jnp.ones((N, 1), jnp.float32)
    if sout is None:
        sout = jnp.ones((N, 1), jnp.float32)
    if x is None:
        x = jnp.zeros((1, D_IN), jnp.float32)
    return pl.pallas_call(
        functools.partial(_dense_body, tail),
        out_shape=out_shape,
    )(part, sin, sout, L['W'], vec, x)


def _prep_body(xm_ref, x_ref, s1_ref, s2_ref, t1_ref, t2_ref):
    t1_ref[:N, :] = xm_ref[...] * s1_ref[...]
    t1_ref[N:, :] = jnp.zeros((NZPAD, D_IN), jnp.float32)
    t2_ref[:N, :] = x_ref[...] * s2_ref[...]
    t2_ref[N:, :] = jnp.zeros((NZPAD, D_IN), jnp.float32)


def _prep(xm, x, s1, s2):
    return pl.pallas_call(
        _prep_body,
        out_shape=[jax.ShapeDtypeStruct((NP, D_IN), jnp.float32),
                   jax.ShapeDtypeStruct((NP, D_IN), jnp.float32)],
    )(xm, x, s1, s2)


_BI = 400


def _loss2_body(hi_ref, h_ref, a_ref, out_ref):
    p = lax.dot_general(hi_ref[...], h_ref[...], (((1,), (1,)), ((), ())),
                        preferred_element_type=jnp.float32)
    d = p - a_ref[...]
    part = jnp.sum(d * d).reshape(1, 1)

    @pl.when(pl.program_id(0) == 0)
    def _():
        out_ref[...] = jnp.zeros((1, 1), jnp.float32)

    out_ref[...] += part


def _loss2(h2, matrix):
    gi = N // _BI
    total = pl.pallas_call(
        _loss2_body,
        grid=(gi,),
        in_specs=[
            pl.BlockSpec((_BI, D_IN), lambda i: (i, 0)),
            pl.BlockSpec((N, D_IN), lambda i: (0, 0)),
            pl.BlockSpec((_BI, N), lambda i: (i, 0)),
        ],
        out_specs=pl.BlockSpec((1, 1), lambda i: (0, 0)),
        out_shape=jax.ShapeDtypeStruct((1, 1), jnp.float32),
    )(h2, h2, matrix)
    return total[0, 0] / (float(N) * float(N))


def kernel(x, edge_index, matrix, params):
    src, dst = edge_index[0].astype(jnp.int32), edge_index[1].astype(jnp.int32)
    msrc, mdst, mw = _mask_edges_fast(src, dst)

    token = jnp.asarray(_TOKEN_FLAG)[:, None]
    xm = jnp.where(token, params['mask_token'][0][None, :], x[jnp.asarray(_SRCMAP)])

    deg_o1 = jnp.clip(jnp.zeros((N,), jnp.float32).at[src].add(1.0), 1.0, None)
    deg_i1 = jnp.clip(jnp.zeros((N,), jnp.float32).at[dst].add(1.0), 1.0, None)
    deg_o2 = jnp.clip(jnp.zeros((N,), jnp.float32).at[msrc].add(mw), 1.0, None)
    deg_i2 = jnp.clip(jnp.zeros((N,), jnp.float32).at[mdst].add(mw), 1.0, None)
    so1 = (deg_o1 ** -0.5)[:, None]
    si1 = (deg_i1 ** -0.5)[:, None]
    so2 = (deg_o2 ** -0.5)[:, None]
    si2 = (deg_i2 ** -0.5)[:, None]

    sx1, dx1 = _pad_edges(src, dst, EP1)
    active = mw > 0.0
    msrc_r = jnp.where(active, msrc,
                       N + (jnp.arange(E2, dtype=jnp.int32) & (NZPAD - 1)))
    sx2, dx2 = _pad_edges(msrc_r, mdst, EP2)

    enc1, enc2 = params['enc']

    t1, t2 = _prep(xm, x, so1, so2)

    p = _agg(t1, sx1, dx1, NB1)
    h = _dense('scale_pad', p, si1, so1, enc1)
    p = _agg(h, sx1, dx1, NB1)
    h = _dense('pad', p, si1, None, enc2)
    p = _agg(h, sx1, dx1, NB1)
    loss1 = _dense('sce', p, None, None, params['dec1'][0], x=x)[0, 0]

    p = _agg(t2, sx2, dx2, NB2)
    h = _dense('scale_pad', p, si2, so2, enc1)
    p = _agg(h, sx2, dx2, NB2)
    h = _dense('pad', p, si2, None, enc2)
    p = _agg(h, sx2, dx2, NB2)
    h2n = _dense('l2norm', p, None, None, params['dec2'][0])

    loss2 = _loss2(h2n, matrix)
    return ALPHA * loss1 + (1.0 - ALPHA) * loss2
